# baseline (device time: 46194 ns/iter reference)
import jax
import jax.numpy as jnp
from jax import lax
from jax.experimental import pallas as pl
from jax.experimental.pallas import tpu as pltpu

N_DEV = 4


def kernel(x):
    m, n = x.shape
    TN = 256
    nt = n // TN

    def body(x_ref, o_ref, own_ref, tot_ref, ssems, rsems):
        i = pl.program_id(0)
        my = lax.axis_index("i")

        barrier = pltpu.get_barrier_semaphore()

        @pl.when(i == 0)
        def _():
            for d in range(1, N_DEV):
                pl.semaphore_signal(
                    barrier,
                    inc=1,
                    device_id=((my + d) % N_DEV,),
                    device_id_type=pl.DeviceIdType.MESH,
                )
            pl.semaphore_wait(barrier, N_DEV - 1)

        xv = x_ref[...]

        C, R = m // 64, 64
        s = jnp.log(xv).reshape(C, R, TN)
        row_i = lax.broadcasted_iota(jnp.int32, (R, R), 0)
        col_i = lax.broadcasted_iota(jnp.int32, (R, R), 1)
        ltri = (row_i >= col_i).astype(jnp.bfloat16)
        ltri_b = jnp.broadcast_to(ltri, (C, R, R))
        S = lax.dot_general(
            ltri_b,
            s.astype(jnp.bfloat16),
            dimension_numbers=(((2,), (1,)), ((0,), (0,))),
            preferred_element_type=jnp.float32,
        )

        cs = jnp.sum(s, axis=1)
        row_c = lax.broadcasted_iota(jnp.int32, (C, C), 0)
        col_c = lax.broadcasted_iota(jnp.int32, (C, C), 1)
        inc = lax.dot_general(
            (row_c >= col_c).astype(jnp.float32),
            cs,
            dimension_numbers=(((1,), (0,)), ((), ())),
            preferred_element_type=jnp.float32,
        )

        own_ref[pl.ds(i, 1), :] = inc[C - 1 : C, :]
        rdmas = []
        for d in range(1, N_DEV):
            rdma = pltpu.make_async_remote_copy(
                src_ref=own_ref.at[pl.ds(i, 1)],
                dst_ref=tot_ref.at[pl.ds(i * (N_DEV - 1) + d - 1, 1)],
                send_sem=ssems.at[i * (N_DEV - 1) + d - 1],
                recv_sem=rsems.at[i * (N_DEV - 1) + d - 1],
                device_id=((my + d) % N_DEV,),
                device_id_type=pl.DeviceIdType.MESH,
            )
            rdma.start()
            rdmas.append(rdma)

        exc = jnp.concatenate(
            [jnp.zeros((1, TN), jnp.float32), inc[: C - 1]], axis=0
        )

        pref = jnp.zeros((1, TN), jnp.float32)
        for d in range(1, N_DEV):
            rdmas[d - 1].wait_recv()
            src_dev = (my - d) % N_DEV
            tvals = tot_ref[pl.ds(i * (N_DEV - 1) + d - 1, 1), :]
            pref = pref + jnp.where(src_dev < my, tvals, jnp.zeros_like(tvals))
        for d in range(1, N_DEV):
            rdmas[d - 1].wait_send()

        offs = exc + pref
        out = jnp.exp(S + offs[:, None, :])
        o_ref[...] = out.reshape(m, TN).astype(o_ref.dtype)

    return pl.pallas_call(
        body,
        grid=(nt,),
        out_shape=jax.ShapeDtypeStruct((m, n), jnp.bfloat16),
        in_specs=[pl.BlockSpec((m, TN), lambda i: (0, i))],
        out_specs=pl.BlockSpec((m, TN), lambda i: (0, i)),
        scratch_shapes=[
            pltpu.VMEM((nt, TN), jnp.float32),
            pltpu.VMEM((nt * (N_DEV - 1), TN), jnp.float32),
            pltpu.SemaphoreType.DMA((nt * (N_DEV - 1),)),
            pltpu.SemaphoreType.DMA((nt * (N_DEV - 1),)),
        ],
        compiler_params=pltpu.CompilerParams(
            dimension_semantics=("arbitrary",),
            collective_id=0,
            vmem_limit_bytes=100 * 1024 * 1024,
        ),
    )(x)


# device time: 39205 ns/iter; 1.1783x vs baseline; 1.1783x over previous
import jax
import jax.numpy as jnp
from jax import lax
from jax.experimental import pallas as pl
from jax.experimental.pallas import tpu as pltpu

N_DEV = 4


def kernel(x):
    m, n = x.shape
    TN = 256
    nt = n // TN
    C, R = m // 64, 64

    def body(x_ref, o_ref, own_ref, tot_ref, s_scr, e_scr, ssems, rsems):
        i = pl.program_id(0)
        my = lax.axis_index("i")

        barrier = pltpu.get_barrier_semaphore()

        @pl.when(i == 0)
        def _():
            for d in range(1, N_DEV):
                pl.semaphore_signal(
                    barrier,
                    inc=1,
                    device_id=((my + d) % N_DEV,),
                    device_id_type=pl.DeviceIdType.MESH,
                )
            pl.semaphore_wait(barrier, N_DEV - 1)

        @pl.when(i < nt)
        def _():
            s = jnp.log(x_ref[...]).reshape(C, R, TN)
            row_i = lax.broadcasted_iota(jnp.int32, (R, R), 0)
            col_i = lax.broadcasted_iota(jnp.int32, (R, R), 1)
            ltri_b = jnp.broadcast_to(
                (row_i >= col_i).astype(jnp.float32), (C, R, R)
            )
            S = lax.dot_general(
                ltri_b,
                s,
                dimension_numbers=(((2,), (1,)), ((0,), (0,))),
                preferred_element_type=jnp.float32,
            )

            cs = S[:, R - 1 : R, :].reshape(C, TN)
            row_c = lax.broadcasted_iota(jnp.int32, (C, C), 0)
            col_c = lax.broadcasted_iota(jnp.int32, (C, C), 1)
            inc = lax.dot_general(
                (row_c >= col_c).astype(jnp.float32),
                cs,
                dimension_numbers=(((1,), (0,)), ((), ())),
                preferred_element_type=jnp.float32,
            )

            own_ref[pl.ds(i, 1), :] = inc[C - 1 : C, :]
            for d in range(1, N_DEV):
                pltpu.make_async_remote_copy(
                    src_ref=own_ref.at[pl.ds(i, 1)],
                    dst_ref=tot_ref.at[pl.ds(i * (N_DEV - 1) + d - 1, 1)],
                    send_sem=ssems.at[i * (N_DEV - 1) + d - 1],
                    recv_sem=rsems.at[i * (N_DEV - 1) + d - 1],
                    device_id=((my + d) % N_DEV,),
                    device_id_type=pl.DeviceIdType.MESH,
                ).start()

            slot = i % 2
            s_scr[pl.ds(slot, 1)] = S[None]
            e_scr[pl.ds(slot, 1)] = jnp.concatenate(
                [jnp.zeros((1, TN), jnp.float32), inc[: C - 1]], axis=0
            )[None]

        @pl.when(i > 0)
        def _():
            j = i - 1
            pref = jnp.zeros((1, TN), jnp.float32)
            for d in range(1, N_DEV):
                rdma = pltpu.make_async_remote_copy(
                    src_ref=own_ref.at[pl.ds(j, 1)],
                    dst_ref=tot_ref.at[pl.ds(j * (N_DEV - 1) + d - 1, 1)],
                    send_sem=ssems.at[j * (N_DEV - 1) + d - 1],
                    recv_sem=rsems.at[j * (N_DEV - 1) + d - 1],
                    device_id=((my + d) % N_DEV,),
                    device_id_type=pl.DeviceIdType.MESH,
                )
                rdma.wait_recv()
                rdma.wait_send()
                src_dev = (my - d) % N_DEV
                tvals = tot_ref[pl.ds(j * (N_DEV - 1) + d - 1, 1), :]
                pref = pref + jnp.where(
                    src_dev < my, tvals, jnp.zeros_like(tvals)
                )

            slot = j % 2
            offs = e_scr[slot] + pref
            out = jnp.exp(s_scr[slot] + offs[:, None, :])
            o_ref[...] = out.reshape(m, TN).astype(o_ref.dtype)

    return pl.pallas_call(
        body,
        grid=(nt + 1,),
        out_shape=jax.ShapeDtypeStruct((m, n), jnp.bfloat16),
        in_specs=[
            pl.BlockSpec((m, TN), lambda i: (0, jnp.minimum(i, nt - 1)))
        ],
        out_specs=pl.BlockSpec((m, TN), lambda i: (0, jnp.maximum(i - 1, 0))),
        scratch_shapes=[
            pltpu.VMEM((nt, TN), jnp.float32),
            pltpu.VMEM((nt * (N_DEV - 1), TN), jnp.float32),
            pltpu.VMEM((2, C, R, TN), jnp.float32),
            pltpu.VMEM((2, C, TN), jnp.float32),
            pltpu.SemaphoreType.DMA((nt * (N_DEV - 1),)),
            pltpu.SemaphoreType.DMA((nt * (N_DEV - 1),)),
        ],
        compiler_params=pltpu.CompilerParams(
            dimension_semantics=("arbitrary",),
            collective_id=0,
            vmem_limit_bytes=100 * 1024 * 1024,
        ),
    )(x)


# device time: 37258 ns/iter; 1.2398x vs baseline; 1.0523x over previous
import jax
import jax.numpy as jnp
from jax import lax
from jax.experimental import pallas as pl
from jax.experimental.pallas import tpu as pltpu

N_DEV = 4
DEPTH = 2


def kernel(x):
    m, n = x.shape
    TN = 256
    nt = n // TN
    C, R = m // 64, 64
    D = DEPTH
    NS = D + 1

    def body(x_ref, o_ref, own_ref, tot_ref, s_scr, e_scr, ssems, rsems):
        i = pl.program_id(0)
        my = lax.axis_index("i")

        barrier = pltpu.get_barrier_semaphore()

        @pl.when(i == 0)
        def _():
            for d in range(1, N_DEV):
                pl.semaphore_signal(
                    barrier,
                    inc=1,
                    device_id=((my + d) % N_DEV,),
                    device_id_type=pl.DeviceIdType.MESH,
                )

        @pl.when(i < nt)
        def _():
            s = jnp.log(x_ref[...]).reshape(C, R, TN)
            row_i = lax.broadcasted_iota(jnp.int32, (R, R), 0)
            col_i = lax.broadcasted_iota(jnp.int32, (R, R), 1)
            ltri_b = jnp.broadcast_to(
                (row_i >= col_i).astype(jnp.float32), (C, R, R)
            )
            S = lax.dot_general(
                ltri_b,
                s,
                dimension_numbers=(((2,), (1,)), ((0,), (0,))),
                preferred_element_type=jnp.float32,
            )

            cs = S[:, R - 1 : R, :].reshape(C, TN)
            row_c = lax.broadcasted_iota(jnp.int32, (C, C), 0)
            col_c = lax.broadcasted_iota(jnp.int32, (C, C), 1)
            inc = lax.dot_general(
                (row_c >= col_c).astype(jnp.float32),
                cs,
                dimension_numbers=(((1,), (0,)), ((), ())),
                preferred_element_type=jnp.float32,
            )

            own_ref[pl.ds(i, 1), :] = inc[C - 1 : C, :]

            @pl.when(i == 0)
            def _():
                pl.semaphore_wait(barrier, N_DEV - 1)

            for d in range(1, N_DEV):
                pltpu.make_async_remote_copy(
                    src_ref=own_ref.at[pl.ds(i, 1)],
                    dst_ref=tot_ref.at[pl.ds(i * (N_DEV - 1) + d - 1, 1)],
                    send_sem=ssems.at[i * (N_DEV - 1) + d - 1],
                    recv_sem=rsems.at[i * (N_DEV - 1) + d - 1],
                    device_id=((my + d) % N_DEV,),
                    device_id_type=pl.DeviceIdType.MESH,
                ).start()

            slot = i % NS
            s_scr[pl.ds(slot, 1)] = S[None].astype(jnp.bfloat16)
            e_scr[pl.ds(slot, 1)] = jnp.concatenate(
                [jnp.zeros((1, TN), jnp.float32), inc[: C - 1]], axis=0
            )[None]

        @pl.when(i >= D)
        def _():
            j = i - D
            pref = jnp.zeros((1, TN), jnp.float32)
            for d in range(1, N_DEV):
                rdma = pltpu.make_async_remote_copy(
                    src_ref=own_ref.at[pl.ds(j, 1)],
                    dst_ref=tot_ref.at[pl.ds(j * (N_DEV - 1) + d - 1, 1)],
                    send_sem=ssems.at[j * (N_DEV - 1) + d - 1],
                    recv_sem=rsems.at[j * (N_DEV - 1) + d - 1],
                    device_id=((my + d) % N_DEV,),
                    device_id_type=pl.DeviceIdType.MESH,
                )
                rdma.wait_recv()
                rdma.wait_send()
                src_dev = (my - d) % N_DEV
                tvals = tot_ref[pl.ds(j * (N_DEV - 1) + d - 1, 1), :]
                pref = pref + jnp.where(
                    src_dev < my, tvals, jnp.zeros_like(tvals)
                )

            slot = j % NS
            offs = e_scr[slot] + pref
            out = jnp.exp(s_scr[slot].astype(jnp.float32) + offs[:, None, :])
            o_ref[...] = out.reshape(m, TN).astype(o_ref.dtype)

    return pl.pallas_call(
        body,
        grid=(nt + D,),
        out_shape=jax.ShapeDtypeStruct((m, n), jnp.bfloat16),
        in_specs=[
            pl.BlockSpec((m, TN), lambda i: (0, jnp.minimum(i, nt - 1)))
        ],
        out_specs=pl.BlockSpec(
            (m, TN), lambda i: (0, jnp.clip(i - DEPTH, 0, nt - 1))
        ),
        scratch_shapes=[
            pltpu.VMEM((nt, TN), jnp.float32),
            pltpu.VMEM((nt * (N_DEV - 1), TN), jnp.float32),
            pltpu.VMEM((NS, C, R, TN), jnp.bfloat16),
            pltpu.VMEM((NS, C, TN), jnp.float32),
            pltpu.SemaphoreType.DMA((nt * (N_DEV - 1),)),
            pltpu.SemaphoreType.DMA((nt * (N_DEV - 1),)),
        ],
        compiler_params=pltpu.CompilerParams(
            dimension_semantics=("arbitrary",),
            collective_id=0,
            vmem_limit_bytes=100 * 1024 * 1024,
        ),
    )(x)
